# Initial kernel scaffold; baseline (speedup 1.0000x reference)
import jax
import jax.numpy as jnp
from jax.experimental import pallas as pl


def kernel(input_ids, embedding, pos_enc, time_emb, Wqk, Wv, Wo, Wout, bout, ln_g, ln_b, noise):
    raise NotImplementedError("write your pallas kernel here")


# speedup vs baseline: 1.1296x; 1.1296x over previous
import functools

import numpy as np
import jax
import jax.numpy as jnp
from jax import lax
from jax.experimental import pallas as pl
from jax.experimental.pallas import tpu as pltpu
from jax.experimental.pallas import tpu_sc as plsc

_NUM_HEADS = 12
_T = 4
_CHUNK = 128
_S = 2048
_D = 768
_DH = _D // _NUM_HEADS
_NCH = _S // _CHUNK
_S_TILE = 512
_N_STILES = _S // _S_TILE
_V_TILE = 1024

_ALPHAS = tuple(
    float(a) for a in np.clip(np.linspace(0.1, 1.0, _T), 0.1, 0.9).astype(np.float32)
)


def _sc_gather(embedding, ids):
    info = plsc.get_sparse_core_info()
    nw = info.num_cores * info.num_subcores
    b_per_w = _S // nw
    mesh = plsc.VectorSubcoreMesh(core_axis_name="c", subcore_axis_name="s")

    @functools.partial(
        pl.kernel,
        mesh=mesh,
        out_type=jax.ShapeDtypeStruct((_S, _D), jnp.float32),
        scratch_types=[
            pltpu.VMEM((b_per_w,), jnp.int32),
            pltpu.VMEM((b_per_w, _D), jnp.float32),
            pltpu.SemaphoreType.DMA,
        ],
    )
    def gather_kernel(table_hbm, idx_hbm, out_hbm, idx_v, rows_v, sem):
        wid = lax.axis_index("s") * info.num_cores + lax.axis_index("c")
        base = wid * b_per_w
        pltpu.sync_copy(idx_hbm.at[pl.ds(base, b_per_w)], idx_v)
        pltpu.async_copy(table_hbm.at[idx_v], rows_v, sem).wait()
        pltpu.sync_copy(rows_v, out_hbm.at[pl.ds(base, b_per_w)])

    return gather_kernel(embedding, ids)


def _diffusion_body(
    temb_ref, wqk_ref, wv_ref, wo_ref, noise_ref, h0_ref, pos_ref,
    lng_ref, lnb_ref, out_ref, h_s, qk_s, v_s, ao_s,
):
    t = pl.program_id(0)

    @pl.when(t == 0)
    def _init():
        h_s[...] = h0_ref[...] + pos_ref[...]

    def stage_a(i, carry):
        off = i * _S_TILE
        h_tile = h_s[pl.ds(off, _S_TILE), :] + temb_ref[0]
        h_s[pl.ds(off, _S_TILE), :] = h_tile
        hb = h_tile.astype(jnp.bfloat16)
        qk = jnp.dot(hb, wqk_ref[0], preferred_element_type=jnp.float32)
        v = jnp.dot(hb, wv_ref[0], preferred_element_type=jnp.float32)
        qk_s[pl.ds(off, _S_TILE), :] = qk.astype(jnp.bfloat16)
        v_s[pl.ds(off, _S_TILE), :] = v.astype(jnp.bfloat16)
        return carry

    lax.fori_loop(0, _N_STILES, stage_a, 0)

    row = lax.broadcasted_iota(jnp.int32, (_CHUNK, _CHUNK), 0)
    col = lax.broadcasted_iota(jnp.int32, (_CHUNK, _CHUNK), 1)
    mask = jnp.concatenate(
        [
            jnp.zeros((_CHUNK, _CHUNK), jnp.float32),
            jnp.where(row == col, jnp.float32(1e5), jnp.float32(0.0)),
        ],
        axis=1,
    )

    def stage_b(c, carry):
        off = c * _CHUNK
        poff = lax.rem(c + _NCH - 1, _NCH) * _CHUNK
        for h_i in range(_NUM_HEADS):
            cs = h_i * _DH
            q = qk_s[pl.ds(off, _CHUNK), cs:cs + _DH]
            k_prev = qk_s[pl.ds(poff, _CHUNK), cs:cs + _DH]
            kcat = jnp.concatenate([k_prev, q], axis=0).astype(jnp.float32)
            norm = jnp.sqrt(jnp.sum(kcat * kcat, axis=1, keepdims=True))
            khat = (kcat / (norm + 1e-6)).astype(jnp.bfloat16)
            s = lax.dot_general(
                q, khat, (((1,), (1,)), ((), ())),
                preferred_element_type=jnp.float32,
            )
            s = s * jnp.float32(1.0 / 8.0) - mask
            m = jnp.max(s, axis=1, keepdims=True)
            e = jnp.exp(s - m)
            p = (e / jnp.sum(e, axis=1, keepdims=True)).astype(jnp.bfloat16)
            vcat = jnp.concatenate(
                [v_s[pl.ds(poff, _CHUNK), cs:cs + _DH],
                 v_s[pl.ds(off, _CHUNK), cs:cs + _DH]],
                axis=0,
            )
            o = jnp.dot(p, vcat, preferred_element_type=jnp.float32)
            ao_s[pl.ds(off, _CHUNK), cs:cs + _DH] = o.astype(jnp.bfloat16)
        return carry

    lax.fori_loop(0, _NCH, stage_b, 0)

    alpha = jnp.float32(_ALPHAS[0])
    for i in range(1, _T):
        alpha = jnp.where(t == i, jnp.float32(_ALPHAS[i]), alpha)

    def stage_c(i, carry):
        off = i * _S_TILE
        proj = jnp.dot(
            ao_s[pl.ds(off, _S_TILE), :], wo_ref[0],
            preferred_element_type=jnp.float32,
        )
        hh = h_s[pl.ds(off, _S_TILE), :] + proj
        mu = jnp.mean(hh, axis=1, keepdims=True)
        d = hh - mu
        var = jnp.mean(d * d, axis=1, keepdims=True)
        hln = d / jnp.sqrt(var + 1e-5) * lng_ref[...] + lnb_ref[...]
        hn = hln + alpha * noise_ref[0, pl.ds(off, _S_TILE), :]
        h_s[pl.ds(off, _S_TILE), :] = hn
        out_ref[pl.ds(off, _S_TILE), :] = hn
        return carry

    lax.fori_loop(0, _N_STILES, stage_c, 0)


def _diffuse(h0, pos, temb, wqk16, wv16, wo16, noise, lng, lnb):
    return pl.pallas_call(
        _diffusion_body,
        grid=(_T,),
        in_specs=[
            pl.BlockSpec((1, 1, _D), lambda t: (t, 0, 0)),
            pl.BlockSpec((1, _D, _D), lambda t: (t, 0, 0)),
            pl.BlockSpec((1, _D, _D), lambda t: (t, 0, 0)),
            pl.BlockSpec((1, _D, _D), lambda t: (t, 0, 0)),
            pl.BlockSpec((1, _S, _D), lambda t: (t, 0, 0)),
            pl.BlockSpec((_S, _D), lambda t: (0, 0)),
            pl.BlockSpec((_S, _D), lambda t: (0, 0)),
            pl.BlockSpec((1, _D), lambda t: (0, 0)),
            pl.BlockSpec((1, _D), lambda t: (0, 0)),
        ],
        out_specs=pl.BlockSpec((_S, _D), lambda t: (0, 0)),
        out_shape=jax.ShapeDtypeStruct((_S, _D), jnp.float32),
        scratch_shapes=[
            pltpu.VMEM((_S, _D), jnp.float32),
            pltpu.VMEM((_S, _D), jnp.bfloat16),
            pltpu.VMEM((_S, _D), jnp.bfloat16),
            pltpu.VMEM((_S, _D), jnp.bfloat16),
        ],
        compiler_params=pltpu.CompilerParams(
            dimension_semantics=("arbitrary",),
        ),
    )(temb, wqk16, wv16, wo16, noise, h0, pos, lng, lnb)


def _proj_body(h_ref, w_ref, b_ref, out_ref):
    out_ref[...] = (
        jnp.dot(h_ref[...], w_ref[...], preferred_element_type=jnp.float32)
        + b_ref[...]
    )


def _project(h16, w16, b, vocab):
    return pl.pallas_call(
        _proj_body,
        grid=(vocab // _V_TILE,),
        in_specs=[
            pl.BlockSpec((_S, _D), lambda i: (0, 0)),
            pl.BlockSpec((_D, _V_TILE), lambda i: (0, i)),
            pl.BlockSpec((1, _V_TILE), lambda i: (0, i)),
        ],
        out_specs=pl.BlockSpec((_S, _V_TILE), lambda i: (0, i)),
        out_shape=jax.ShapeDtypeStruct((_S, vocab), jnp.float32),
        compiler_params=pltpu.CompilerParams(
            dimension_semantics=("parallel",),
        ),
    )(h16, w16, b)


def kernel(input_ids, embedding, pos_enc, time_emb, Wqk, Wv, Wo, Wout, bout,
           ln_g, ln_b, noise):
    vocab = Wout.shape[1]
    ids = input_ids.reshape(-1).astype(jnp.int32)
    h0 = _sc_gather(embedding, ids)
    h_fin = _diffuse(
        h0,
        pos_enc[:_S],
        time_emb.reshape(_T, 1, _D),
        Wqk.astype(jnp.bfloat16),
        Wv.astype(jnp.bfloat16),
        Wo.astype(jnp.bfloat16),
        noise.reshape(_T, _S, _D),
        ln_g.reshape(1, _D),
        ln_b.reshape(1, _D),
    )
    logits = _project(
        h_fin.astype(jnp.bfloat16),
        Wout.astype(jnp.bfloat16),
        bout.reshape(1, vocab),
        vocab,
    )
    return logits.reshape(1, _S, vocab)


# speedup vs baseline: 2.4780x; 2.1937x over previous
import functools

import numpy as np
import jax
import jax.numpy as jnp
from jax import lax
from jax.experimental import pallas as pl
from jax.experimental.pallas import tpu as pltpu
from jax.experimental.pallas import tpu_sc as plsc

_NUM_HEADS = 12
_T = 4
_CHUNK = 128
_S = 2048
_D = 768
_DH = _D // _NUM_HEADS
_NCH = _S // _CHUNK
_S_TILE = 1024
_N_STILES = _S // _S_TILE
_V_TILE = 1024

_ALPHAS = tuple(
    float(a) for a in np.clip(np.linspace(0.1, 1.0, _T), 0.1, 0.9).astype(np.float32)
)

_BLOCK_DIAG = 64.0 * (
    np.arange(_D)[:, None] // _DH == np.arange(_D)[None, :] // _DH
).astype(np.float32)

_STACK_MASK = 1e4 * (
    np.arange(2 * _CHUNK)[None, :] - _CHUNK
    == np.arange(_NUM_HEADS * _CHUNK)[:, None] % _CHUNK
).astype(np.float32)


def _sc_gather(embedding, ids):
    info = plsc.get_sparse_core_info()
    nw = info.num_cores * info.num_subcores
    b_per_w = _S // nw
    mesh = plsc.VectorSubcoreMesh(core_axis_name="c", subcore_axis_name="s")

    half = b_per_w // 2

    @functools.partial(
        pl.kernel,
        mesh=mesh,
        out_type=jax.ShapeDtypeStruct((_S, _D), jnp.float32),
        scratch_types=[
            pltpu.VMEM((b_per_w,), jnp.int32),
            pltpu.VMEM((half, _D), jnp.float32),
            pltpu.VMEM((half, _D), jnp.float32),
            pltpu.SemaphoreType.DMA,
            pltpu.SemaphoreType.DMA,
        ],
    )
    def gather_kernel(table_hbm, idx_hbm, out_hbm, idx_v, rows_a, rows_b,
                      sem_a, sem_b):
        wid = lax.axis_index("s") * info.num_cores + lax.axis_index("c")
        base = wid * b_per_w
        pltpu.sync_copy(idx_hbm.at[pl.ds(base, b_per_w)], idx_v)
        cp_a = pltpu.async_copy(
            table_hbm.at[idx_v.at[pl.ds(0, half)]], rows_a, sem_a)
        cp_b = pltpu.async_copy(
            table_hbm.at[idx_v.at[pl.ds(half, half)]], rows_b, sem_b)
        cp_a.wait()
        pltpu.sync_copy(rows_a, out_hbm.at[pl.ds(base, half)])
        cp_b.wait()
        pltpu.sync_copy(rows_b, out_hbm.at[pl.ds(base + half, half)])

    return gather_kernel(embedding, ids)


def _diffusion_body(
    temb_ref, wqk_ref, wv_ref, wo_ref, noise_ref, h0_ref, pos_ref,
    lng_ref, lnb_ref, bd_ref, mask_ref, out_ref, h_s, qk_s, kh_s, v_s, ao_s,
):
    t = pl.program_id(0)

    @pl.when(t == 0)
    def _init():
        h_s[...] = h0_ref[...] + pos_ref[...]

    for i in range(_N_STILES):
        off = i * _S_TILE
        h_tile = h_s[pl.ds(off, _S_TILE), :] + temb_ref[0]
        h_s[pl.ds(off, _S_TILE), :] = h_tile
        hb = h_tile.astype(jnp.bfloat16)
        qk = jnp.dot(hb, wqk_ref[0], preferred_element_type=jnp.float32)
        v = jnp.dot(hb, wv_ref[0], preferred_element_type=jnp.float32)
        qk_s[pl.ds(off, _S_TILE), :] = qk.astype(jnp.bfloat16)
        v_s[pl.ds(off, _S_TILE), :] = v.astype(jnp.bfloat16)
        qk2 = (qk * qk).astype(jnp.bfloat16)
        n2 = jnp.dot(qk2, bd_ref[...], preferred_element_type=jnp.float32)
        inv = lax.rsqrt(n2 + jnp.float32(6.4e-11))
        kh_s[pl.ds(off, _S_TILE), :] = (qk * inv).astype(jnp.bfloat16)

    mask_all = mask_ref[...]

    def one_chunk(off, poff):
        q_all = qk_s[pl.ds(off, _CHUNK), :]
        k_all = jnp.concatenate(
            [kh_s[pl.ds(poff, _CHUNK), :], kh_s[pl.ds(off, _CHUNK), :]],
            axis=0,
        )
        v_all = jnp.concatenate(
            [v_s[pl.ds(poff, _CHUNK), :], v_s[pl.ds(off, _CHUNK), :]],
            axis=0,
        )
        s_parts = []
        for h_i in range(_NUM_HEADS):
            cs = h_i * _DH
            s_parts.append(
                lax.dot_general(
                    q_all[:, cs:cs + _DH], k_all[:, cs:cs + _DH],
                    (((1,), (1,)), ((), ())),
                    preferred_element_type=jnp.float32,
                ).astype(jnp.bfloat16)
            )
        s_all = jnp.concatenate(s_parts, axis=0) - mask_all
        e16 = jnp.exp(s_all)
        r_all = jnp.float32(1.0) / jnp.sum(
            e16, axis=1, keepdims=True, dtype=jnp.float32)
        o_parts = []
        for h_i in range(_NUM_HEADS):
            cs = h_i * _DH
            o = jnp.dot(e16[h_i * _CHUNK:(h_i + 1) * _CHUNK, :],
                        v_all[:, cs:cs + _DH],
                        preferred_element_type=jnp.float32)
            o = o * r_all[h_i * _CHUNK:(h_i + 1) * _CHUNK, :]
            o_parts.append(o.astype(jnp.bfloat16))
        ao_s[pl.ds(off, _CHUNK), :] = jnp.concatenate(o_parts, axis=1)

    def stage_b(c, carry):
        off = c * _CHUNK
        one_chunk(off, lax.rem(off + _S - _CHUNK, _S))
        return carry

    lax.fori_loop(0, _NCH, stage_b, 0)

    alpha = jnp.float32(_ALPHAS[0])
    for i in range(1, _T):
        alpha = jnp.where(t == i, jnp.float32(_ALPHAS[i]), alpha)

    for i in range(_N_STILES):
        off = i * _S_TILE
        proj = jnp.dot(
            ao_s[pl.ds(off, _S_TILE), :], wo_ref[0],
            preferred_element_type=jnp.float32,
        )
        hh = h_s[pl.ds(off, _S_TILE), :] + proj
        mu = jnp.mean(hh, axis=1, keepdims=True)
        d = hh - mu
        var = jnp.mean(d * d, axis=1, keepdims=True)
        hln = d / jnp.sqrt(var + 1e-5) * lng_ref[...] + lnb_ref[...]
        hn = hln + alpha * noise_ref[0, pl.ds(off, _S_TILE), :]
        h_s[pl.ds(off, _S_TILE), :] = hn
        out_ref[pl.ds(off, _S_TILE), :] = hn.astype(jnp.bfloat16)


def _diffuse(h0, pos, temb, wqk16, wv16, wo16, noise, lng, lnb, bd16, mask2):
    return pl.pallas_call(
        _diffusion_body,
        grid=(_T,),
        in_specs=[
            pl.BlockSpec((1, 1, _D), lambda t: (t, 0, 0)),
            pl.BlockSpec((1, _D, _D), lambda t: (t, 0, 0)),
            pl.BlockSpec((1, _D, _D), lambda t: (t, 0, 0)),
            pl.BlockSpec((1, _D, _D), lambda t: (t, 0, 0)),
            pl.BlockSpec((1, _S, _D), lambda t: (t, 0, 0)),
            pl.BlockSpec((_S, _D), lambda t: (0, 0)),
            pl.BlockSpec((_S, _D), lambda t: (0, 0)),
            pl.BlockSpec((1, _D), lambda t: (0, 0)),
            pl.BlockSpec((1, _D), lambda t: (0, 0)),
            pl.BlockSpec((_D, _D), lambda t: (0, 0)),
            pl.BlockSpec((_NUM_HEADS * _CHUNK, 2 * _CHUNK), lambda t: (0, 0)),
        ],
        out_specs=pl.BlockSpec((_S, _D), lambda t: (0, 0)),
        out_shape=jax.ShapeDtypeStruct((_S, _D), jnp.bfloat16),
        scratch_shapes=[
            pltpu.VMEM((_S, _D), jnp.float32),
            pltpu.VMEM((_S, _D), jnp.bfloat16),
            pltpu.VMEM((_S, _D), jnp.bfloat16),
            pltpu.VMEM((_S, _D), jnp.bfloat16),
            pltpu.VMEM((_S, _D), jnp.bfloat16),
        ],
        compiler_params=pltpu.CompilerParams(
            dimension_semantics=("arbitrary",),
            vmem_limit_bytes=64 * 1024 * 1024,
        ),
    )(temb, wqk16, wv16, wo16, noise, h0, pos, lng, lnb, bd16, mask2)


def _proj_body(h_ref, w_ref, b_ref, out_ref):
    out_ref[...] = (
        jnp.dot(h_ref[...], w_ref[...].astype(jnp.bfloat16),
                preferred_element_type=jnp.float32)
        + b_ref[...]
    )


def _project(h16, w16, b, vocab):
    return pl.pallas_call(
        _proj_body,
        grid=(vocab // _V_TILE,),
        in_specs=[
            pl.BlockSpec((_S, _D), lambda i: (0, 0)),
            pl.BlockSpec((_D, _V_TILE), lambda i: (0, i)),
            pl.BlockSpec((1, _V_TILE), lambda i: (0, i)),
        ],
        out_specs=pl.BlockSpec((_S, _V_TILE), lambda i: (0, i)),
        out_shape=jax.ShapeDtypeStruct((_S, vocab), jnp.float32),
        compiler_params=pltpu.CompilerParams(
            dimension_semantics=("parallel",),
        ),
    )(h16, w16, b)


def kernel(input_ids, embedding, pos_enc, time_emb, Wqk, Wv, Wo, Wout, bout,
           ln_g, ln_b, noise):
    vocab = Wout.shape[1]
    ids = input_ids.reshape(-1).astype(jnp.int32)
    h0 = _sc_gather(embedding, ids)
    h_fin = _diffuse(
        h0,
        pos_enc[:_S],
        time_emb.reshape(_T, 1, _D),
        Wqk.astype(jnp.bfloat16),
        Wv.astype(jnp.bfloat16),
        Wo.astype(jnp.bfloat16),
        noise.reshape(_T, _S, _D),
        ln_g.reshape(1, _D),
        ln_b.reshape(1, _D),
        jnp.asarray(_BLOCK_DIAG, dtype=jnp.bfloat16),
        jnp.asarray(_STACK_MASK, dtype=jnp.bfloat16),
    )
    logits = _project(h_fin, Wout, bout.reshape(1, vocab), vocab)
    return logits.reshape(1, _S, vocab)
